# baseline (speedup 1.0000x reference)
import jax
import jax.numpy as jnp
from jax.experimental import pallas as pl
from jax.experimental.pallas import tpu as pltpu

B, H, W, C = 8, 24, 24, 384
E = 8
HW = H * W
CC = C * C
NEG = -1e30
GRP = 4


def _moe_kernel(x_ref, wg_ref, bg_ref, be_ref, we_hbm,
                out_ref, laux_ref,
                wescr_ref, bcomb_ref, wcomb_ref, sem):
    n = pl.program_id(0)

    @pl.when(n == 0)
    def _gate_and_combine():
        copy_a = pltpu.make_async_copy(we_hbm.at[pl.ds(0, E // 2)],
                                       wescr_ref.at[pl.ds(0, E // 2)],
                                       sem.at[0])
        copy_b = pltpu.make_async_copy(we_hbm.at[pl.ds(E // 2, E // 2)],
                                       wescr_ref.at[pl.ds(E // 2, E // 2)],
                                       sem.at[1])
        copy_a.start()
        copy_b.start()

        xall = x_ref[...].reshape(B * HW, C)
        r = jax.lax.broadcasted_iota(jnp.int32, (B, B * HW), 0)
        c = jax.lax.broadcasted_iota(jnp.int32, (B, B * HW), 1)
        avg = jnp.where((c >= r * HW) & (c < (r + 1) * HW),
                        jnp.float32(1.0 / HW), jnp.float32(0.0))
        pooled = jnp.dot(avg, xall, preferred_element_type=jnp.float32)

        logits = jnp.dot(pooled, wg_ref[...],
                         preferred_element_type=jnp.float32) + bg_ref[...]
        m = jnp.max(logits, axis=1, keepdims=True)
        eg = jnp.exp(logits - m)
        gates = eg / jnp.sum(eg, axis=1, keepdims=True)

        iota = jax.lax.broadcasted_iota(jnp.int32, (B, E), 1)
        m1 = jnp.max(gates, axis=1, keepdims=True)
        i1 = jnp.min(jnp.where(gates == m1, iota, E), axis=1, keepdims=True)
        mask1 = (iota == i1)
        g2 = jnp.where(mask1, NEG, gates)
        m2 = jnp.max(g2, axis=1, keepdims=True)
        i2 = jnp.min(jnp.where(g2 == m2, iota, E), axis=1, keepdims=True)

        e2 = jnp.exp(m2 - m1)
        coeff = (jnp.where(mask1, 1.0, 0.0)
                 + jnp.where(iota == i2, e2, 0.0)) / (1.0 + e2)
        bcomb_ref[...] = jnp.dot(coeff, be_ref[...],
                                 preferred_element_type=jnp.float32)

        me = jnp.mean(gates, axis=0, keepdims=True)
        ce = jnp.mean(mask1.astype(jnp.float32), axis=0, keepdims=True)
        laux_ref[...] = jnp.sum(me * ce, axis=1, keepdims=True) * E

        cbf = coeff.astype(jnp.bfloat16)
        copy_a.wait()
        wfa = wescr_ref[pl.ds(0, E // 2)].astype(jnp.bfloat16
                                                 ).reshape(E // 2, CC)
        acc = jnp.dot(cbf[:, :E // 2], wfa,
                      preferred_element_type=jnp.float32)
        copy_b.wait()
        wfb = wescr_ref[pl.ds(E // 2, E // 2)].astype(jnp.bfloat16
                                                      ).reshape(E // 2, CC)
        acc = acc + jnp.dot(cbf[:, E // 2:], wfb,
                            preferred_element_type=jnp.float32)
        wcomb_ref[...] = acc.astype(jnp.bfloat16).reshape(B, C, C)

    for i in range(GRP):
        img = n * GRP + i
        x_n = x_ref[pl.ds(img, 1)].reshape(HW, C).astype(jnp.bfloat16)
        w_n = wcomb_ref[pl.ds(img, 1)].reshape(C, C)
        y = jax.lax.dot_general(x_n, w_n, (((1,), (1,)), ((), ())),
                                preferred_element_type=jnp.float32)
        y = y + bcomb_ref[pl.ds(img, 1)]
        out_ref[i:i + 1] = y.reshape(1, H, W, C)


def kernel(inputs_raw, W_gate, b_gate, W_experts, b_experts):
    bg = b_gate.reshape(1, E)

    out, laux = pl.pallas_call(
        _moe_kernel,
        grid=(B // GRP,),
        in_specs=[
            pl.BlockSpec((B, H, W, C), lambda n: (0, 0, 0, 0)),
            pl.BlockSpec((C, E), lambda n: (0, 0)),
            pl.BlockSpec((1, E), lambda n: (0, 0)),
            pl.BlockSpec((E, C), lambda n: (0, 0)),
            pl.BlockSpec(memory_space=pltpu.MemorySpace.HBM),
        ],
        out_specs=(
            pl.BlockSpec((GRP, H, W, C), lambda n: (n, 0, 0, 0)),
            pl.BlockSpec((1, 1), lambda n: (0, 0)),
        ),
        out_shape=(
            jax.ShapeDtypeStruct((B, H, W, C), jnp.float32),
            jax.ShapeDtypeStruct((1, 1), jnp.float32),
        ),
        scratch_shapes=[
            pltpu.VMEM((E, C, C), jnp.float32),
            pltpu.VMEM((B, C), jnp.float32),
            pltpu.VMEM((B, C, C), jnp.bfloat16),
            pltpu.SemaphoreType.DMA((2,)),
        ],
    )(inputs_raw, W_gate, bg, b_experts, W_experts)

    return out, laux[0, 0]


# speedup vs baseline: 1.0922x; 1.0922x over previous
import jax
import jax.numpy as jnp
from jax.experimental import pallas as pl
from jax.experimental.pallas import tpu as pltpu

B, H, W, C = 8, 24, 24, 384
E = 8
HW = H * W
CC = C * C
NEG = -1e30
GRP = 4


def _moe_kernel(x_ref, wg_ref, bg_ref, be_ref, we_hbm,
                out_ref, laux_ref,
                wescr_ref, bcomb_ref, wcomb_ref, sem):
    n = pl.program_id(0)

    @pl.when(n == 0)
    def _gate_and_combine():
        copy = pltpu.make_async_copy(we_hbm, wescr_ref, sem)
        copy.start()

        xall = x_ref[...].reshape(B * HW, C)
        r = jax.lax.broadcasted_iota(jnp.int32, (B, B * HW), 0)
        c = jax.lax.broadcasted_iota(jnp.int32, (B, B * HW), 1)
        avg = jnp.where((c >= r * HW) & (c < (r + 1) * HW),
                        jnp.float32(1.0 / HW), jnp.float32(0.0))
        pooled = jnp.dot(avg, xall, preferred_element_type=jnp.float32)

        logits = jnp.dot(pooled, wg_ref[...],
                         preferred_element_type=jnp.float32) + bg_ref[...]
        m = jnp.max(logits, axis=1, keepdims=True)
        eg = jnp.exp(logits - m)
        gates = eg / jnp.sum(eg, axis=1, keepdims=True)

        iota = jax.lax.broadcasted_iota(jnp.int32, (B, E), 1)
        m1 = jnp.max(gates, axis=1, keepdims=True)
        i1 = jnp.min(jnp.where(gates == m1, iota, E), axis=1, keepdims=True)
        mask1 = (iota == i1)
        g2 = jnp.where(mask1, NEG, gates)
        m2 = jnp.max(g2, axis=1, keepdims=True)
        i2 = jnp.min(jnp.where(g2 == m2, iota, E), axis=1, keepdims=True)

        e2 = jnp.exp(m2 - m1)
        coeff = (jnp.where(mask1, 1.0, 0.0)
                 + jnp.where(iota == i2, e2, 0.0)) / (1.0 + e2)
        bcomb_ref[...] = jnp.dot(coeff, be_ref[...],
                                 preferred_element_type=jnp.float32)

        me = jnp.mean(gates, axis=0, keepdims=True)
        ce = jnp.mean(mask1.astype(jnp.float32), axis=0, keepdims=True)
        laux_ref[...] = jnp.sum(me * ce, axis=1, keepdims=True) * E

        copy.wait()
        we_flat = wescr_ref[...].astype(jnp.bfloat16).reshape(E, CC)
        wcomb_ref[...] = jnp.dot(coeff.astype(jnp.bfloat16), we_flat,
                                 preferred_element_type=jnp.float32
                                 ).astype(jnp.bfloat16).reshape(B, C, C)

    for i in range(GRP):
        img = n * GRP + i
        x_n = x_ref[pl.ds(img, 1)].reshape(HW, C).astype(jnp.bfloat16)
        w_n = wcomb_ref[pl.ds(img, 1)].reshape(C, C)
        y = jax.lax.dot_general(x_n, w_n, (((1,), (1,)), ((), ())),
                                preferred_element_type=jnp.float32)
        y = y + bcomb_ref[pl.ds(img, 1)]
        out_ref[i:i + 1] = y.reshape(1, H, W, C)


def kernel(inputs_raw, W_gate, b_gate, W_experts, b_experts):
    bg = b_gate.reshape(1, E)

    out, laux = pl.pallas_call(
        _moe_kernel,
        grid=(B // GRP,),
        in_specs=[
            pl.BlockSpec((B, H, W, C), lambda n: (0, 0, 0, 0)),
            pl.BlockSpec((C, E), lambda n: (0, 0)),
            pl.BlockSpec((1, E), lambda n: (0, 0)),
            pl.BlockSpec((E, C), lambda n: (0, 0)),
            pl.BlockSpec(memory_space=pltpu.MemorySpace.HBM),
        ],
        out_specs=(
            pl.BlockSpec((GRP, H, W, C), lambda n: (n, 0, 0, 0)),
            pl.BlockSpec((1, 1), lambda n: (0, 0)),
        ),
        out_shape=(
            jax.ShapeDtypeStruct((B, H, W, C), jnp.float32),
            jax.ShapeDtypeStruct((1, 1), jnp.float32),
        ),
        scratch_shapes=[
            pltpu.VMEM((E, C, C), jnp.float32),
            pltpu.VMEM((B, C), jnp.float32),
            pltpu.VMEM((B, C, C), jnp.bfloat16),
            pltpu.SemaphoreType.DMA,
        ],
    )(inputs_raw, W_gate, bg, b_experts, W_experts)

    return out, laux[0, 0]
